# Initial kernel scaffold; baseline (speedup 1.0000x reference)
import functools

import jax
import jax.numpy as jnp
from jax import lax
from jax.experimental import pallas as pl
from jax.experimental.pallas import tpu as pltpu
from jax.experimental.pallas import tpu_sc as plsc

NC = 2
NS = 16
NW = NC * NS
CH = 128


def _sc_mesh():
  return plsc.VectorSubcoreMesh(core_axis_name="c", subcore_axis_name="s")


def _make_deg_kernel(npad, epw):
  rpt = npad // NS
  w = 16

  @functools.partial(
      pl.kernel,
      out_type=jax.ShapeDtypeStruct((NC * npad, w), jnp.float32),
      mesh=_sc_mesh(),
      scratch_types=[
          pltpu.VMEM((CH,), jnp.int32),
          pltpu.VMEM((CH, w), jnp.float32),
          pltpu.VMEM_SHARED((npad, w), jnp.float32),
      ],
  )
  def deg_kernel(dst_hbm, ones_hbm, zrow_hbm, out_hbm, dst_v, ones_v, acc):
    cid = lax.axis_index("c")
    sid = lax.axis_index("s")
    wid = sid * NC + cid
    pltpu.sync_copy(zrow_hbm, acc.at[pl.ds(sid * rpt, rpt)])
    pltpu.sync_copy(ones_hbm, ones_v)
    plsc.subcore_barrier()

    def body(i, carry):
      base = pl.multiple_of(wid * epw + i * CH, CH)
      pltpu.sync_copy(dst_hbm.at[pl.ds(base, CH)], dst_v)
      pltpu.sync_copy(ones_v, acc.at[dst_v], add=True)
      return carry

    lax.fori_loop(0, epw // CH, body, 0)
    plsc.subcore_barrier()
    pltpu.sync_copy(acc.at[pl.ds(sid * rpt, rpt)],
                    out_hbm.at[pl.ds(cid * npad + sid * rpt, rpt)])

  return deg_kernel


def _make_agg_kernel(npad, epw, w):
  rpt = npad // NS

  @functools.partial(
      pl.kernel,
      out_type=jax.ShapeDtypeStruct((NC * npad, w), jnp.float32),
      mesh=_sc_mesh(),
      scratch_types=[
          pltpu.VMEM((CH,), jnp.int32),
          pltpu.VMEM((CH,), jnp.int32),
          pltpu.VMEM((CH, w), jnp.float32),
          pltpu.VMEM_SHARED((npad, w), jnp.float32),
          pltpu.SemaphoreType.DMA,
      ],
  )
  def agg_kernel(src_hbm, dst_hbm, g_hbm, zrow_hbm, out_hbm,
                 src_v, dst_v, rows_v, acc, sem):
    cid = lax.axis_index("c")
    sid = lax.axis_index("s")
    wid = sid * NC + cid
    pltpu.sync_copy(zrow_hbm, acc.at[pl.ds(sid * rpt, rpt)])
    plsc.subcore_barrier()

    def body(i, carry):
      base = pl.multiple_of(wid * epw + i * CH, CH)
      pltpu.sync_copy(src_hbm.at[pl.ds(base, CH)], src_v)
      pltpu.sync_copy(dst_hbm.at[pl.ds(base, CH)], dst_v)
      pltpu.async_copy(g_hbm.at[src_v], rows_v, sem).wait()
      pltpu.sync_copy(rows_v, acc.at[dst_v], add=True)
      return carry

    lax.fori_loop(0, epw // CH, body, 0)
    plsc.subcore_barrier()
    pltpu.sync_copy(acc.at[pl.ds(sid * rpt, rpt)],
                    out_hbm.at[pl.ds(cid * npad + sid * rpt, rpt)])

  return agg_kernel


def _matmul_xw0(x, w0, bm):
  n, d = x.shape
  h = w0.shape[1]

  def body(x_ref, w_ref, o_ref):
    o_ref[...] = jnp.dot(x_ref[...], w_ref[...],
                         preferred_element_type=jnp.float32)

  return pl.pallas_call(
      body,
      grid=(n // bm,),
      in_specs=[
          pl.BlockSpec((bm, d), lambda i: (i, 0)),
          pl.BlockSpec((d, h), lambda i: (0, 0)),
      ],
      out_specs=pl.BlockSpec((bm, h), lambda i: (i, 0)),
      out_shape=jax.ShapeDtypeStruct((n, h), jnp.float32),
  )(x, w0)


def _prep_g0(degp, xw0, bm):
  n, h = xw0.shape

  def body(d_ref, x_ref, dinv_ref, g_ref):
    deg = d_ref[0, :, 0:1] + d_ref[1, :, 0:1] + 1.0
    dinv = lax.rsqrt(deg)
    dinv_b = jnp.broadcast_to(dinv, (bm, h))
    dinv_ref[...] = dinv_b
    g_ref[...] = dinv_b * x_ref[...]

  return pl.pallas_call(
      body,
      grid=(n // bm,),
      in_specs=[
          pl.BlockSpec((2, bm, 16), lambda i: (0, i, 0)),
          pl.BlockSpec((bm, h), lambda i: (i, 0)),
      ],
      out_specs=[
          pl.BlockSpec((bm, h), lambda i: (i, 0)),
          pl.BlockSpec((bm, h), lambda i: (i, 0)),
      ],
      out_shape=[
          jax.ShapeDtypeStruct((n, h), jnp.float32),
          jax.ShapeDtypeStruct((n, h), jnp.float32),
      ],
  )(degp, xw0)


def _h1_heads(s1p, xw0, dinv, wc, bm):
  n, h = xw0.shape

  def body(s_ref, x_ref, dv_ref, w_ref, c_ref, g_ref):
    dinv = dv_ref[...]
    agg = dinv * (s_ref[0] + s_ref[1]) + dinv * dinv * x_ref[...]
    h1 = jnp.maximum(agg, 0.0)
    c = jnp.dot(h1, w_ref[...], preferred_element_type=jnp.float32)
    c_ref[...] = c
    g_ref[...] = dinv * c

  return pl.pallas_call(
      body,
      grid=(n // bm,),
      in_specs=[
          pl.BlockSpec((2, bm, h), lambda i: (0, i, 0)),
          pl.BlockSpec((bm, h), lambda i: (i, 0)),
          pl.BlockSpec((bm, h), lambda i: (i, 0)),
          pl.BlockSpec((h, h), lambda i: (0, 0)),
      ],
      out_specs=[
          pl.BlockSpec((bm, h), lambda i: (i, 0)),
          pl.BlockSpec((bm, h), lambda i: (i, 0)),
      ],
      out_shape=[
          jax.ShapeDtypeStruct((n, h), jnp.float32),
          jax.ShapeDtypeStruct((n, h), jnp.float32),
      ],
  )(s1p, xw0, dinv, wc)


def _reparam(s2p, c, dinv, eps, bm):
  n, h = c.shape
  h2 = h // 2

  def body(s_ref, c_ref, dv_ref, e_ref, z_ref):
    dinv = dv_ref[...]
    zc = dinv * (s_ref[0] + s_ref[1]) + dinv * dinv * c_ref[...]
    zm = zc[:, :h2]
    zl = zc[:, h2:]
    z_ref[...] = zm + e_ref[...] * jnp.exp(zl)

  return pl.pallas_call(
      body,
      grid=(n // bm,),
      in_specs=[
          pl.BlockSpec((2, bm, h), lambda i: (0, i, 0)),
          pl.BlockSpec((bm, h), lambda i: (i, 0)),
          pl.BlockSpec((bm, h), lambda i: (i, 0)),
          pl.BlockSpec((bm, h2), lambda i: (i, 0)),
      ],
      out_specs=pl.BlockSpec((bm, h2), lambda i: (i, 0)),
      out_shape=jax.ShapeDtypeStruct((n, h2), jnp.float32),
  )(s2p, c, dinv, eps)


def _decoder(z, zt, bm, bn):
  n, k = z.shape

  def body(a_ref, b_ref, o_ref):
    o_ref[...] = jnp.dot(a_ref[...], b_ref[...],
                         preferred_element_type=jnp.float32)

  return pl.pallas_call(
      body,
      grid=(n // bm, n // bn),
      in_specs=[
          pl.BlockSpec((bm, k), lambda i, j: (i, 0)),
          pl.BlockSpec((k, bn), lambda i, j: (0, j)),
      ],
      out_specs=pl.BlockSpec((bm, bn), lambda i, j: (i, j)),
      out_shape=jax.ShapeDtypeStruct((n, n), jnp.float32),
  )(z, zt)


def kernel(x, edge_index, W0, W1, W2, eps):
  n = x.shape[0]
  e = edge_index.shape[1]

  npad = ((n + NS * 8 - 1) // (NS * 8)) * (NS * 8)
  epw = ((e + NW * CH - 1) // (NW * CH)) * CH
  epad = NW * epw

  pad = epad - e
  src = jnp.concatenate([edge_index[0], jnp.full((pad,), n, jnp.int32)])
  dst = jnp.concatenate([edge_index[1], jnp.full((pad,), n, jnp.int32)])

  ones_blk = jnp.ones((CH, 16), jnp.float32)
  zrow16 = jnp.zeros((npad // NS, 16), jnp.float32)
  zrow32 = jnp.zeros((npad // NS, 32), jnp.float32)

  deg_k = _make_deg_kernel(npad, epw)
  agg_k = _make_agg_kernel(npad, epw, 32)

  degp = deg_k(dst, ones_blk, zrow16).reshape(NC, npad, 16)

  xw0 = _matmul_xw0(x, W0, bm=1000)
  dinv, g0 = _prep_g0(degp, xw0, bm=1000)

  g0p = jnp.pad(g0, ((0, npad - n), (0, 0)))
  s1p = agg_k(src, dst, g0p, zrow32).reshape(NC, npad, 32)

  wc = jnp.concatenate([W1, W2], axis=1)
  c, g1 = _h1_heads(s1p, xw0, dinv, wc, bm=1000)

  g1p = jnp.pad(g1, ((0, npad - n), (0, 0)))
  s2p = agg_k(src, dst, g1p, zrow32).reshape(NC, npad, 32)

  z = _reparam(s2p, c, dinv, eps, bm=1000)

  recon = _decoder(z, z.T, bm=500, bn=2000)
  return recon.reshape(-1)


# speedup vs baseline: 11.4775x; 11.4775x over previous
import functools

import jax
import jax.numpy as jnp
from jax import lax
from jax.experimental import pallas as pl
from jax.experimental.pallas import tpu as pltpu
from jax.experimental.pallas import tpu_sc as plsc

NC = 2
NS = 16
NW = NC * NS
CH = 128


def _sc_mesh():
  return plsc.VectorSubcoreMesh(core_axis_name="c", subcore_axis_name="s")


def _make_deg_kernel(npad, epw):
  rpt = npad // NS
  w = 16

  @functools.partial(
      pl.kernel,
      out_type=jax.ShapeDtypeStruct((NC * npad, w), jnp.float32),
      mesh=_sc_mesh(),
      scratch_types=[
          pltpu.VMEM((CH,), jnp.int32),
          pltpu.VMEM((CH, w), jnp.float32),
          pltpu.VMEM_SHARED((npad, w), jnp.float32),
      ],
      compiler_params=pltpu.CompilerParams(use_tc_tiling_on_sc=False),
  )
  def deg_kernel(dst_hbm, ones_hbm, zrow_hbm, out_hbm, dst_v, ones_v, acc):
    cid = lax.axis_index("c")
    sid = lax.axis_index("s")
    wid = sid * NC + cid
    pltpu.sync_copy(zrow_hbm, acc.at[pl.ds(sid * rpt, rpt)])
    pltpu.sync_copy(ones_hbm, ones_v)
    plsc.subcore_barrier()

    def body(i, carry):
      base = pl.multiple_of(wid * epw + i * CH, CH)
      pltpu.sync_copy(dst_hbm.at[pl.ds(base, CH)], dst_v)
      pltpu.sync_copy(ones_v, acc.at[dst_v], add=True)
      return carry

    lax.fori_loop(0, epw // CH, body, 0)
    plsc.subcore_barrier()
    pltpu.sync_copy(acc.at[pl.ds(sid * rpt, rpt)],
                    out_hbm.at[pl.ds(cid * npad + sid * rpt, rpt)])

  return deg_kernel


def _make_agg_kernel(npad, epw, w):
  rpt = npad // NS

  @functools.partial(
      pl.kernel,
      out_type=jax.ShapeDtypeStruct((NC * npad, w), jnp.float32),
      mesh=_sc_mesh(),
      scratch_types=[
          pltpu.VMEM((CH,), jnp.int32),
          pltpu.VMEM((CH,), jnp.int32),
          pltpu.VMEM((CH, w), jnp.float32),
          pltpu.VMEM_SHARED((npad, w), jnp.float32),
          pltpu.SemaphoreType.DMA,
      ],
      compiler_params=pltpu.CompilerParams(use_tc_tiling_on_sc=False),
  )
  def agg_kernel(src_hbm, dst_hbm, g_hbm, zrow_hbm, out_hbm,
                 src_v, dst_v, rows_v, acc, sem):
    cid = lax.axis_index("c")
    sid = lax.axis_index("s")
    wid = sid * NC + cid
    pltpu.sync_copy(zrow_hbm, acc.at[pl.ds(sid * rpt, rpt)])
    plsc.subcore_barrier()

    def body(i, carry):
      base = pl.multiple_of(wid * epw + i * CH, CH)
      pltpu.sync_copy(src_hbm.at[pl.ds(base, CH)], src_v)
      pltpu.sync_copy(dst_hbm.at[pl.ds(base, CH)], dst_v)
      pltpu.async_copy(g_hbm.at[src_v], rows_v, sem).wait()
      pltpu.sync_copy(rows_v, acc.at[dst_v], add=True)
      return carry

    lax.fori_loop(0, epw // CH, body, 0)
    plsc.subcore_barrier()
    pltpu.sync_copy(acc.at[pl.ds(sid * rpt, rpt)],
                    out_hbm.at[pl.ds(cid * npad + sid * rpt, rpt)])

  return agg_kernel


def _matmul_xw0(x, w0, bm):
  n, d = x.shape
  h = w0.shape[1]

  def body(x_ref, w_ref, o_ref):
    o_ref[...] = jnp.dot(x_ref[...], w_ref[...],
                         preferred_element_type=jnp.float32)

  return pl.pallas_call(
      body,
      grid=(n // bm,),
      in_specs=[
          pl.BlockSpec((bm, d), lambda i: (i, 0)),
          pl.BlockSpec((d, h), lambda i: (0, 0)),
      ],
      out_specs=pl.BlockSpec((bm, h), lambda i: (i, 0)),
      out_shape=jax.ShapeDtypeStruct((n, h), jnp.float32),
  )(x, w0)


def _prep_g0(degp, xw0, bm):
  n, h = xw0.shape

  def body(d_ref, x_ref, dinv_ref, g_ref):
    deg = d_ref[0, :, 0:1] + d_ref[1, :, 0:1] + 1.0
    dinv = lax.rsqrt(deg)
    dinv_b = jnp.broadcast_to(dinv, (bm, h))
    dinv_ref[...] = dinv_b
    g_ref[...] = dinv_b * x_ref[...]

  return pl.pallas_call(
      body,
      grid=(n // bm,),
      in_specs=[
          pl.BlockSpec((2, bm, 16), lambda i: (0, i, 0)),
          pl.BlockSpec((bm, h), lambda i: (i, 0)),
      ],
      out_specs=[
          pl.BlockSpec((bm, h), lambda i: (i, 0)),
          pl.BlockSpec((bm, h), lambda i: (i, 0)),
      ],
      out_shape=[
          jax.ShapeDtypeStruct((n, h), jnp.float32),
          jax.ShapeDtypeStruct((n, h), jnp.float32),
      ],
  )(degp, xw0)


def _h1_heads(s1p, xw0, dinv, wc, bm):
  n, h = xw0.shape

  def body(s_ref, x_ref, dv_ref, w_ref, c_ref, g_ref):
    dinv = dv_ref[...]
    agg = dinv * (s_ref[0] + s_ref[1]) + dinv * dinv * x_ref[...]
    h1 = jnp.maximum(agg, 0.0)
    c = jnp.dot(h1, w_ref[...], preferred_element_type=jnp.float32)
    c_ref[...] = c
    g_ref[...] = dinv * c

  return pl.pallas_call(
      body,
      grid=(n // bm,),
      in_specs=[
          pl.BlockSpec((2, bm, h), lambda i: (0, i, 0)),
          pl.BlockSpec((bm, h), lambda i: (i, 0)),
          pl.BlockSpec((bm, h), lambda i: (i, 0)),
          pl.BlockSpec((h, h), lambda i: (0, 0)),
      ],
      out_specs=[
          pl.BlockSpec((bm, h), lambda i: (i, 0)),
          pl.BlockSpec((bm, h), lambda i: (i, 0)),
      ],
      out_shape=[
          jax.ShapeDtypeStruct((n, h), jnp.float32),
          jax.ShapeDtypeStruct((n, h), jnp.float32),
      ],
  )(s1p, xw0, dinv, wc)


def _reparam(s2p, c, dinv, eps, bm):
  n, h = c.shape
  h2 = h // 2

  def body(s_ref, c_ref, dv_ref, e_ref, z_ref):
    dinv = dv_ref[...]
    zc = dinv * (s_ref[0] + s_ref[1]) + dinv * dinv * c_ref[...]
    zm = zc[:, :h2]
    zl = zc[:, h2:]
    z_ref[...] = zm + e_ref[...] * jnp.exp(zl)

  return pl.pallas_call(
      body,
      grid=(n // bm,),
      in_specs=[
          pl.BlockSpec((2, bm, h), lambda i: (0, i, 0)),
          pl.BlockSpec((bm, h), lambda i: (i, 0)),
          pl.BlockSpec((bm, h), lambda i: (i, 0)),
          pl.BlockSpec((bm, h2), lambda i: (i, 0)),
      ],
      out_specs=pl.BlockSpec((bm, h2), lambda i: (i, 0)),
      out_shape=jax.ShapeDtypeStruct((n, h2), jnp.float32),
  )(s2p, c, dinv, eps)


def _decoder(z, zt, bm, bn):
  n, k = z.shape

  def body(a_ref, b_ref, o_ref):
    o_ref[...] = jnp.dot(a_ref[...], b_ref[...],
                         preferred_element_type=jnp.float32)

  return pl.pallas_call(
      body,
      grid=(pl.cdiv(n, bm), pl.cdiv(n, bn)),
      in_specs=[
          pl.BlockSpec((bm, k), lambda i, j: (i, 0)),
          pl.BlockSpec((k, bn), lambda i, j: (0, j)),
      ],
      out_specs=pl.BlockSpec((bm, bn), lambda i, j: (i, j)),
      out_shape=jax.ShapeDtypeStruct((n, n), jnp.float32),
  )(z, zt)


def kernel(x, edge_index, W0, W1, W2, eps):
  n = x.shape[0]
  e = edge_index.shape[1]

  npad = ((n + NS * 8 - 1) // (NS * 8)) * (NS * 8)
  epw = ((e + NW * CH - 1) // (NW * CH)) * CH
  epad = NW * epw

  pad = epad - e
  src = jnp.concatenate([edge_index[0], jnp.full((pad,), n, jnp.int32)])
  dst = jnp.concatenate([edge_index[1], jnp.full((pad,), n, jnp.int32)])

  ones_blk = jnp.ones((CH, 16), jnp.float32)
  zrow16 = jnp.zeros((npad // NS, 16), jnp.float32)
  zrow32 = jnp.zeros((npad // NS, 32), jnp.float32)

  deg_k = _make_deg_kernel(npad, epw)
  agg_k = _make_agg_kernel(npad, epw, 32)

  degp = deg_k(dst, ones_blk, zrow16).reshape(NC, npad, 16)

  xw0 = _matmul_xw0(x, W0, bm=1000)
  dinv, g0 = _prep_g0(degp, xw0, bm=1000)

  g0p = jnp.pad(g0, ((0, npad - n), (0, 0)))
  s1p = agg_k(src, dst, g0p, zrow32).reshape(NC, npad, 32)

  wc = jnp.concatenate([W1, W2], axis=1)
  c, g1 = _h1_heads(s1p, xw0, dinv, wc, bm=1000)

  g1p = jnp.pad(g1, ((0, npad - n), (0, 0)))
  s2p = agg_k(src, dst, g1p, zrow32).reshape(NC, npad, 32)

  z = _reparam(s2p, c, dinv, eps, bm=1000)

  recon = _decoder(z, z.T, bm=400, bn=2048)
  return recon.reshape(-1)


# speedup vs baseline: 13.6832x; 1.1922x over previous
import functools

import jax
import jax.numpy as jnp
from jax import lax
from jax.experimental import pallas as pl
from jax.experimental.pallas import tpu as pltpu
from jax.experimental.pallas import tpu_sc as plsc

NC = 2
NS = 16
NW = NC * NS
CH = 128
K = 8


def _sc_mesh():
  return plsc.VectorSubcoreMesh(core_axis_name="c", subcore_axis_name="s")


def _make_deg_kernel(npad, epw):
  rpt = npad // NS
  w = 16
  nslabs = epw // (K * CH)
  assert nslabs % 2 == 0 and nslabs >= 2

  @functools.partial(
      pl.kernel,
      out_type=jax.ShapeDtypeStruct((NC * npad, w), jnp.float32),
      mesh=_sc_mesh(),
      scratch_types=[
          pltpu.VMEM((K, CH), jnp.int32),
          pltpu.VMEM((K, CH), jnp.int32),
          pltpu.VMEM((CH, w), jnp.float32),
          pltpu.VMEM_SHARED((npad, w), jnp.float32),
          pltpu.SemaphoreType.DMA,
          pltpu.SemaphoreType.DMA,
      ],
      compiler_params=pltpu.CompilerParams(use_tc_tiling_on_sc=False),
  )
  def deg_kernel(dst2_hbm, ones_hbm, zrow_hbm, out_hbm,
                 dst_s0, dst_s1, ones_v, acc, ssem0, ssem1):
    cid = lax.axis_index("c")
    sid = lax.axis_index("s")
    wid = sid * NC + cid
    r0 = wid * (epw // CH)

    pltpu.sync_copy(zrow_hbm, acc.at[pl.ds(sid * rpt, rpt)])
    pltpu.sync_copy(ones_hbm, ones_v)
    plsc.subcore_barrier()

    def load(buf, s):
      pltpu.sync_copy(dst2_hbm.at[pl.ds(r0 + s * K, K)], buf)

    def fire(buf, sem):
      for k in range(K):
        pltpu.async_copy(ones_v, acc.at[buf.at[k]], sem, add=True)

    def drain(sem):
      for _ in range(K):
        pltpu.make_async_copy(ones_hbm, ones_v, sem).wait()

    load(dst_s0, 0)

    def body(j, carry):
      s = 2 * j
      fire(dst_s0, ssem0)

      @pl.when(j > 0)
      def _():
        drain(ssem1)

      load(dst_s1, s + 1)
      fire(dst_s1, ssem1)
      drain(ssem0)

      @pl.when(j + 1 < nslabs // 2)
      def _():
        load(dst_s0, s + 2)

      return carry

    lax.fori_loop(0, nslabs // 2, body, 0)
    drain(ssem1)
    plsc.subcore_barrier()
    pltpu.sync_copy(acc.at[pl.ds(sid * rpt, rpt)],
                    out_hbm.at[pl.ds(cid * npad + sid * rpt, rpt)])

  return deg_kernel


def _make_agg_kernel(npad, epw, w):
  rpt = npad // NS
  nslabs = epw // (K * CH)
  assert nslabs % 2 == 0 and nslabs >= 2

  @functools.partial(
      pl.kernel,
      out_type=jax.ShapeDtypeStruct((NC * npad, w), jnp.float32),
      mesh=_sc_mesh(),
      scratch_types=[
          pltpu.VMEM((K, CH), jnp.int32),
          pltpu.VMEM((K, CH), jnp.int32),
          pltpu.VMEM((K, CH), jnp.int32),
          pltpu.VMEM((K, CH), jnp.int32),
          pltpu.VMEM((K * CH, w), jnp.float32),
          pltpu.VMEM((K * CH, w), jnp.float32),
          pltpu.VMEM_SHARED((npad, w), jnp.float32),
          pltpu.SemaphoreType.DMA,
          pltpu.SemaphoreType.DMA,
          pltpu.SemaphoreType.DMA,
          pltpu.SemaphoreType.DMA,
      ],
      compiler_params=pltpu.CompilerParams(use_tc_tiling_on_sc=False),
  )
  def agg_kernel(src2_hbm, dst2_hbm, g_hbm, zrow_hbm, out_hbm,
                 src_s0, src_s1, dst_s0, dst_s1, rows0, rows1, acc,
                 gsem0, gsem1, ssem0, ssem1):
    cid = lax.axis_index("c")
    sid = lax.axis_index("s")
    wid = sid * NC + cid
    r0 = wid * (epw // CH)

    pltpu.sync_copy(zrow_hbm, acc.at[pl.ds(sid * rpt, rpt)])
    plsc.subcore_barrier()

    def fire_gathers(sbuf, dbuf, rows, gsem, s):
      pltpu.sync_copy(src2_hbm.at[pl.ds(r0 + s * K, K)], sbuf)
      pltpu.sync_copy(dst2_hbm.at[pl.ds(r0 + s * K, K)], dbuf)
      for k in range(K):
        pltpu.async_copy(g_hbm.at[sbuf.at[k]],
                         rows.at[pl.ds(k * CH, CH)], gsem)

    def drain_gathers(rows, gsem):
      pltpu.make_async_copy(g_hbm.at[pl.ds(0, K * CH)], rows, gsem).wait()

    def fire_scatters(dbuf, rows, ssem):
      for k in range(K):
        pltpu.async_copy(rows.at[pl.ds(k * CH, CH)],
                         acc.at[dbuf.at[k]], ssem, add=True)

    def drain_scatters(rows, ssem):
      pltpu.make_async_copy(g_hbm.at[pl.ds(0, K * CH)], rows, ssem).wait()

    fire_gathers(src_s0, dst_s0, rows0, gsem0, 0)

    def body(j, carry):
      s = 2 * j

      @pl.when(j > 0)
      def _():
        drain_scatters(rows1, ssem1)

      fire_gathers(src_s1, dst_s1, rows1, gsem1, s + 1)
      drain_gathers(rows0, gsem0)
      fire_scatters(dst_s0, rows0, ssem0)
      drain_scatters(rows0, ssem0)

      @pl.when(j + 1 < nslabs // 2)
      def _():
        fire_gathers(src_s0, dst_s0, rows0, gsem0, s + 2)

      drain_gathers(rows1, gsem1)
      fire_scatters(dst_s1, rows1, ssem1)
      return carry

    lax.fori_loop(0, nslabs // 2, body, 0)
    drain_scatters(rows1, ssem1)
    plsc.subcore_barrier()
    pltpu.sync_copy(acc.at[pl.ds(sid * rpt, rpt)],
                    out_hbm.at[pl.ds(cid * npad + sid * rpt, rpt)])

  return agg_kernel


def _matmul_xw0(x, w0, bm):
  n, d = x.shape
  h = w0.shape[1]

  def body(x_ref, w_ref, o_ref):
    o_ref[...] = jnp.dot(x_ref[...], w_ref[...],
                         preferred_element_type=jnp.float32)

  return pl.pallas_call(
      body,
      grid=(n // bm,),
      in_specs=[
          pl.BlockSpec((bm, d), lambda i: (i, 0)),
          pl.BlockSpec((d, h), lambda i: (0, 0)),
      ],
      out_specs=pl.BlockSpec((bm, h), lambda i: (i, 0)),
      out_shape=jax.ShapeDtypeStruct((n, h), jnp.float32),
  )(x, w0)


def _prep_g0(degp, xw0, bm):
  n, h = xw0.shape

  def body(d_ref, x_ref, dinv_ref, g_ref):
    deg = d_ref[0, :, 0:1] + d_ref[1, :, 0:1] + 1.0
    dinv = lax.rsqrt(deg)
    dinv_b = jnp.broadcast_to(dinv, (bm, h))
    dinv_ref[...] = dinv_b
    g_ref[...] = dinv_b * x_ref[...]

  return pl.pallas_call(
      body,
      grid=(n // bm,),
      in_specs=[
          pl.BlockSpec((2, bm, 16), lambda i: (0, i, 0)),
          pl.BlockSpec((bm, h), lambda i: (i, 0)),
      ],
      out_specs=[
          pl.BlockSpec((bm, h), lambda i: (i, 0)),
          pl.BlockSpec((bm, h), lambda i: (i, 0)),
      ],
      out_shape=[
          jax.ShapeDtypeStruct((n, h), jnp.float32),
          jax.ShapeDtypeStruct((n, h), jnp.float32),
      ],
  )(degp, xw0)


def _h1_heads(s1p, xw0, dinv, wc, bm):
  n, h = xw0.shape

  def body(s_ref, x_ref, dv_ref, w_ref, c_ref, g_ref):
    dinv = dv_ref[...]
    agg = dinv * (s_ref[0] + s_ref[1]) + dinv * dinv * x_ref[...]
    h1 = jnp.maximum(agg, 0.0)
    c = jnp.dot(h1, w_ref[...], preferred_element_type=jnp.float32)
    c_ref[...] = c
    g_ref[...] = dinv * c

  return pl.pallas_call(
      body,
      grid=(n // bm,),
      in_specs=[
          pl.BlockSpec((2, bm, h), lambda i: (0, i, 0)),
          pl.BlockSpec((bm, h), lambda i: (i, 0)),
          pl.BlockSpec((bm, h), lambda i: (i, 0)),
          pl.BlockSpec((h, h), lambda i: (0, 0)),
      ],
      out_specs=[
          pl.BlockSpec((bm, h), lambda i: (i, 0)),
          pl.BlockSpec((bm, h), lambda i: (i, 0)),
      ],
      out_shape=[
          jax.ShapeDtypeStruct((n, h), jnp.float32),
          jax.ShapeDtypeStruct((n, h), jnp.float32),
      ],
  )(s1p, xw0, dinv, wc)


def _reparam(s2p, c, dinv, eps, bm):
  n, h = c.shape
  h2 = h // 2

  def body(s_ref, c_ref, dv_ref, e_ref, z_ref):
    dinv = dv_ref[...]
    zc = dinv * (s_ref[0] + s_ref[1]) + dinv * dinv * c_ref[...]
    zm = zc[:, :h2]
    zl = zc[:, h2:]
    z_ref[...] = zm + e_ref[...] * jnp.exp(zl)

  return pl.pallas_call(
      body,
      grid=(n // bm,),
      in_specs=[
          pl.BlockSpec((2, bm, h), lambda i: (0, i, 0)),
          pl.BlockSpec((bm, h), lambda i: (i, 0)),
          pl.BlockSpec((bm, h), lambda i: (i, 0)),
          pl.BlockSpec((bm, h2), lambda i: (i, 0)),
      ],
      out_specs=pl.BlockSpec((bm, h2), lambda i: (i, 0)),
      out_shape=jax.ShapeDtypeStruct((n, h2), jnp.float32),
  )(s2p, c, dinv, eps)


def _decoder(z, zt, bm, bn):
  n, k = z.shape

  def body(a_ref, b_ref, o_ref):
    o_ref[...] = jnp.dot(a_ref[...], b_ref[...],
                         preferred_element_type=jnp.float32)

  return pl.pallas_call(
      body,
      grid=(pl.cdiv(n, bm), pl.cdiv(n, bn)),
      in_specs=[
          pl.BlockSpec((bm, k), lambda i, j: (i, 0)),
          pl.BlockSpec((k, bn), lambda i, j: (0, j)),
      ],
      out_specs=pl.BlockSpec((bm, bn), lambda i, j: (i, j)),
      out_shape=jax.ShapeDtypeStruct((n, n), jnp.float32),
  )(z, zt)


def kernel(x, edge_index, W0, W1, W2, eps):
  n = x.shape[0]
  e = edge_index.shape[1]

  npad = ((n + NS * 8 - 1) // (NS * 8)) * (NS * 8)
  slab = 2 * K * CH
  epw = ((e + NW * slab - 1) // (NW * slab)) * slab
  epad = NW * epw

  pad = epad - e
  src = jnp.concatenate([edge_index[0], jnp.full((pad,), n, jnp.int32)])
  dst = jnp.concatenate([edge_index[1], jnp.full((pad,), n, jnp.int32)])
  src = src.reshape(-1, CH)
  dst = dst.reshape(-1, CH)

  ones_blk = jnp.ones((CH, 16), jnp.float32)
  zrow16 = jnp.zeros((npad // NS, 16), jnp.float32)
  zrow32 = jnp.zeros((npad // NS, 32), jnp.float32)

  deg_k = _make_deg_kernel(npad, epw)
  agg_k = _make_agg_kernel(npad, epw, 32)

  degp = deg_k(dst, ones_blk, zrow16).reshape(NC, npad, 16)

  xw0 = _matmul_xw0(x, W0, bm=1000)
  dinv, g0 = _prep_g0(degp, xw0, bm=1000)

  g0p = jnp.pad(g0, ((0, npad - n), (0, 0)))
  s1p = agg_k(src, dst, g0p, zrow32).reshape(NC, npad, 32)

  wc = jnp.concatenate([W1, W2], axis=1)
  c, g1 = _h1_heads(s1p, xw0, dinv, wc, bm=1000)

  g1p = jnp.pad(g1, ((0, npad - n), (0, 0)))
  s2p = agg_k(src, dst, g1p, zrow32).reshape(NC, npad, 32)

  z = _reparam(s2p, c, dinv, eps, bm=1000)

  recon = _decoder(z, z.T, bm=400, bn=2048)
  return recon.reshape(-1)
